# baseline (speedup 1.0000x reference)
import jax
import jax.numpy as jnp
import numpy as np
from jax import lax
from jax.experimental import pallas as pl
from jax.experimental.pallas import tpu as pltpu
from jax.experimental.pallas import tpu_sc as plsc

N_T, N_Y, N_X = 8, 32, 32
NB = N_X * N_Y
OFFS = (-64, -33, -32, -31, -2, -1, 0, 1, 2, 31, 32, 33, 64)
S = (-32, -1, 0, 1, 32)
N_BLK = 3 * N_T - 2
N_FLAT = 2 * N_BLK
N_PAIR = len(S) * len(S)

SC_CORES = 2
SC_SUBCORES = 16
SC_LANES = 16
N_WORK = SC_CORES * SC_SUBCORES

PER_STEP = 2
SUB = 128


def _np_masks():
    k = np.arange(NB)
    x = k % N_X
    y = k // N_X
    me = ((x + 1) < N_X).astype(np.float32)
    mw = ((x - 1) >= 0).astype(np.float32)
    mn = ((y + 1) < N_Y).astype(np.float32)
    ms = ((y - 1) >= 0).astype(np.float32)
    return me, mw, mn, ms


_ME, _MW, _MN, _MS = _np_masks()


def _shift_lanes(v, o):
    if o == 0:
        return v
    z = jnp.zeros((1, abs(o)), v.dtype)
    if o > 0:
        return jnp.concatenate([z, v[:, : NB - o]], axis=1)
    return jnp.concatenate([v[:, -o:], z], axis=1)


def _sc_products_body(a_hbm, b_hbm, w_hbm, p_hbm, a_v, b_v, w_v, p_v):
    wid = lax.axis_index("s") * SC_CORES + lax.axis_index("c")

    def do_block(blk):
        pltpu.sync_copy(a_hbm.at[blk], a_v)
        pltpu.sync_copy(b_hbm.at[blk], b_v)
        pltpu.sync_copy(w_hbm.at[blk], w_v)

        def chunk(c, carry):
            sl = pl.ds(c * SC_LANES, SC_LANES)
            wv = w_v[sl]
            for i1 in range(len(S)):
                aw = a_v[i1, sl] * wv
                for i2 in range(len(S)):
                    p_v[i1 * len(S) + i2, sl] = aw * b_v[i2, sl]
            return carry

        lax.fori_loop(0, NB // SC_LANES, chunk, 0)
        pltpu.sync_copy(p_v, p_hbm.at[blk])

    do_block(wid)

    @pl.when(wid + N_WORK < N_FLAT)
    def _():
        do_block(wid + N_WORK)


def _sc_products(a_flat, b_flat, w_flat):
    mesh = plsc.VectorSubcoreMesh(
        core_axis_name="c", subcore_axis_name="s",
        num_cores=SC_CORES, num_subcores=SC_SUBCORES,
    )
    return pl.kernel(
        _sc_products_body,
        out_type=jax.ShapeDtypeStruct((N_FLAT, N_PAIR, NB), jnp.float32),
        mesh=mesh,
        scratch_types=[
            pltpu.VMEM((len(S), NB), jnp.float32),
            pltpu.VMEM((len(S), NB), jnp.float32),
            pltpu.VMEM((NB,), jnp.float32),
            pltpu.VMEM((N_PAIR, NB), jnp.float32),
        ],
    )(a_flat, b_flat, w_flat)


def _band_kernel(p_ref, e_ref, out_ref):
  for kk in range(PER_STEP):
    dd = {d: None for d in OFFS}
    for i1, o1 in enumerate(S):
        for i2, o2 in enumerate(S):
            row = i1 * len(S) + i2
            term = _shift_lanes(p_ref[0, kk, row : row + 1, :], o1)
            d = o2 - o1
            dd[d] = term if dd[d] is None else dd[d] + term
    dd[0] = dd[0] + e_ref[0, kk]
    g = {d: _shift_lanes(dd[d], d) for d in OFFS}
    nsub = NB // SUB
    jr = jax.lax.broadcasted_iota(jnp.int32, (SUB, SUB), 1) - jax.lax.broadcasted_iota(
        jnp.int32, (SUB, SUB), 0
    )
    zero = jnp.zeros((SUB, SUB), jnp.float32)
    for sr in range(nsub):
        for sc in range(nsub):
            delta = sc - sr
            if abs(delta) > 1:
                out_ref[0, kk, sr * SUB : (sr + 1) * SUB, sc * SUB : (sc + 1) * SUB] = zero
                continue
            acc = zero
            for d in OFFS:
                c = d - SUB * delta
                if c <= -SUB or c >= SUB:
                    continue
                gd = g[d][:, sc * SUB : (sc + 1) * SUB]
                acc = jnp.where(jr == c, jnp.broadcast_to(gd, (SUB, SUB)), acc)
            out_ref[0, kk, sr * SUB : (sr + 1) * SUB, sc * SUB : (sc + 1) * SUB] = acc


def kernel(kappa, m, H, tau):
    del H
    kap = kappa[0]
    t = jnp.squeeze(tau, axis=1)
    qt = jnp.transpose(1.0 / (t * t), (0, 2, 1))
    m1 = jnp.transpose(m[:, 0], (0, 2, 1))
    m2 = jnp.transpose(m[:, 1], (0, 2, 1))
    u1 = 0.5 * m1 * _ME
    l1 = -0.5 * m1 * _MW
    u32 = 0.5 * m2 * _MN
    l32 = -0.5 * m2 * _MS
    k2 = kap * kap
    dvec = jnp.concatenate(
        [jnp.full((2, 1, NB), k2), jnp.full((2, N_T - 1, NB), 1.0 + k2)], axis=1
    )
    Md = jnp.stack([l32, l1, dvec, u1, u32], axis=2)

    ones = jnp.ones((2, NB), jnp.float32)
    zcol = jnp.zeros((2, NB), jnp.float32)
    e0 = jnp.zeros((2, 5, NB), jnp.float32).at[:, 2, :].set(1.0)

    A_l, B_l, W_l, E_l = [], [], [], []

    def add(a, b, w, e):
        A_l.append(a)
        B_l.append(b)
        W_l.append(w)
        E_l.append(e)

    add(Md[:, 0], Md[:, 0], ones, 1.05 * ones)
    add(e0, Md[:, 1], -qt[:, 1], zcol)
    for i in range(1, N_T - 1):
        add(Md[:, i], e0, -qt[:, i], zcol)
        add(Md[:, i], Md[:, i], qt[:, i], qt[:, i])
        add(e0, Md[:, i + 1], -qt[:, i + 1], zcol)
    add(Md[:, N_T - 1], e0, -qt[:, N_T - 1], zcol)
    add(Md[:, N_T - 1], Md[:, N_T - 1], qt[:, N_T - 1], zcol)

    A = jnp.stack(A_l, axis=1)
    B = jnp.stack(B_l, axis=1)
    W = jnp.stack(W_l, axis=1)
    E = jnp.stack(E_l, axis=1)[:, :, None, :]

    P = _sc_products(
        A.reshape(N_FLAT, len(S), NB),
        B.reshape(N_FLAT, len(S), NB),
        W.reshape(N_FLAT, NB),
    ).reshape(2, N_BLK, N_PAIR, NB)

    return pl.pallas_call(
        _band_kernel,
        grid=(2, N_BLK // PER_STEP),
        in_specs=[
            pl.BlockSpec((1, PER_STEP, N_PAIR, NB), lambda b, k: (b, k, 0, 0)),
            pl.BlockSpec((1, PER_STEP, 1, NB), lambda b, k: (b, k, 0, 0)),
        ],
        out_specs=pl.BlockSpec((1, PER_STEP, NB, NB), lambda b, k: (b, k, 0, 0)),
        out_shape=jax.ShapeDtypeStruct((2, N_BLK, NB, NB), jnp.float32),
        compiler_params=pltpu.CompilerParams(
            dimension_semantics=("parallel", "parallel")
        ),
    )(P, E)


# speedup vs baseline: 1.0075x; 1.0075x over previous
import jax
import jax.numpy as jnp
import numpy as np
from jax import lax
from jax.experimental import pallas as pl
from jax.experimental.pallas import tpu as pltpu
from jax.experimental.pallas import tpu_sc as plsc

N_T, N_Y, N_X = 8, 32, 32
NB = N_X * N_Y
OFFS = (-64, -33, -32, -31, -2, -1, 0, 1, 2, 31, 32, 33, 64)
S = (-32, -1, 0, 1, 32)
N_BLK = 3 * N_T - 2
N_FLAT = 2 * N_BLK
N_PAIR = len(S) * len(S)

SC_CORES = 2
SC_SUBCORES = 16
SC_LANES = 16
N_WORK = SC_CORES * SC_SUBCORES

PER_STEP = 2
SUB = 128


def _np_masks():
    k = np.arange(NB)
    x = k % N_X
    y = k // N_X
    me = ((x + 1) < N_X).astype(np.float32)
    mw = ((x - 1) >= 0).astype(np.float32)
    mn = ((y + 1) < N_Y).astype(np.float32)
    ms = ((y - 1) >= 0).astype(np.float32)
    return me, mw, mn, ms


_ME, _MW, _MN, _MS = _np_masks()


def _shift_lanes(v, o):
    if o == 0:
        return v
    z = jnp.zeros((1, abs(o)), v.dtype)
    if o > 0:
        return jnp.concatenate([z, v[:, : NB - o]], axis=1)
    return jnp.concatenate([v[:, -o:], z], axis=1)


def _sc_products_body(abw_hbm, p_hbm, abw_v, p_v):
    wid = lax.axis_index("s") * SC_CORES + lax.axis_index("c")

    def do_block(blk):
        pltpu.sync_copy(abw_hbm.at[blk], abw_v)
        for c in range(NB // SC_LANES):
            sl = pl.ds(c * SC_LANES, SC_LANES)
            wv = abw_v[2 * len(S), sl]
            for i1 in range(len(S)):
                aw = abw_v[i1, sl] * wv
                for i2 in range(len(S)):
                    p_v[i1 * len(S) + i2, sl] = aw * abw_v[len(S) + i2, sl]
        pltpu.sync_copy(p_v, p_hbm.at[blk])

    do_block(wid)

    @pl.when(wid + N_WORK < N_FLAT)
    def _():
        do_block(wid + N_WORK)


def _sc_products(abw_flat):
    mesh = plsc.VectorSubcoreMesh(
        core_axis_name="c", subcore_axis_name="s",
        num_cores=SC_CORES, num_subcores=SC_SUBCORES,
    )
    return pl.kernel(
        _sc_products_body,
        out_type=jax.ShapeDtypeStruct((N_FLAT, N_PAIR, NB), jnp.float32),
        mesh=mesh,
        scratch_types=[
            pltpu.VMEM((2 * len(S) + 1, NB), jnp.float32),
            pltpu.VMEM((N_PAIR, NB), jnp.float32),
        ],
    )(abw_flat)


def _band_kernel(p_ref, e_ref, out_ref):
  for kk in range(PER_STEP):
    dd = {d: None for d in OFFS}
    for i1, o1 in enumerate(S):
        for i2, o2 in enumerate(S):
            row = i1 * len(S) + i2
            term = _shift_lanes(p_ref[0, kk, row : row + 1, :], o1)
            d = o2 - o1
            dd[d] = term if dd[d] is None else dd[d] + term
    dd[0] = dd[0] + e_ref[0, kk]
    g = {d: _shift_lanes(dd[d], d) for d in OFFS}
    nsub = NB // SUB
    jr = jax.lax.broadcasted_iota(jnp.int32, (SUB, SUB), 1) - jax.lax.broadcasted_iota(
        jnp.int32, (SUB, SUB), 0
    )
    zero = jnp.zeros((SUB, SUB), jnp.float32)
    for sr in range(nsub):
        for sc in range(nsub):
            delta = sc - sr
            if abs(delta) > 1:
                out_ref[0, kk, sr * SUB : (sr + 1) * SUB, sc * SUB : (sc + 1) * SUB] = zero
                continue
            acc = zero
            for d in OFFS:
                c = d - SUB * delta
                if c <= -SUB or c >= SUB:
                    continue
                gd = g[d][:, sc * SUB : (sc + 1) * SUB]
                acc = jnp.where(jr == c, jnp.broadcast_to(gd, (SUB, SUB)), acc)
            out_ref[0, kk, sr * SUB : (sr + 1) * SUB, sc * SUB : (sc + 1) * SUB] = acc


def kernel(kappa, m, H, tau):
    del H
    kap = kappa[0]
    t = jnp.squeeze(tau, axis=1)
    qt = jnp.transpose(1.0 / (t * t), (0, 2, 1))
    m1 = jnp.transpose(m[:, 0], (0, 2, 1))
    m2 = jnp.transpose(m[:, 1], (0, 2, 1))
    u1 = 0.5 * m1 * _ME
    l1 = -0.5 * m1 * _MW
    u32 = 0.5 * m2 * _MN
    l32 = -0.5 * m2 * _MS
    k2 = kap * kap
    dvec = jnp.concatenate(
        [jnp.full((2, 1, NB), k2), jnp.full((2, N_T - 1, NB), 1.0 + k2)], axis=1
    )
    Md = jnp.stack([l32, l1, dvec, u1, u32], axis=2)

    ones = jnp.ones((2, NB), jnp.float32)
    zcol = jnp.zeros((2, NB), jnp.float32)
    e0 = jnp.zeros((2, 5, NB), jnp.float32).at[:, 2, :].set(1.0)

    A_l, B_l, W_l, E_l = [], [], [], []

    def add(a, b, w, e):
        A_l.append(a)
        B_l.append(b)
        W_l.append(w)
        E_l.append(e)

    add(Md[:, 0], Md[:, 0], ones, 1.05 * ones)
    add(e0, Md[:, 1], -qt[:, 1], zcol)
    for i in range(1, N_T - 1):
        add(Md[:, i], e0, -qt[:, i], zcol)
        add(Md[:, i], Md[:, i], qt[:, i], qt[:, i])
        add(e0, Md[:, i + 1], -qt[:, i + 1], zcol)
    add(Md[:, N_T - 1], e0, -qt[:, N_T - 1], zcol)
    add(Md[:, N_T - 1], Md[:, N_T - 1], qt[:, N_T - 1], zcol)

    A = jnp.stack(A_l, axis=1)
    B = jnp.stack(B_l, axis=1)
    W = jnp.stack(W_l, axis=1)
    E = jnp.stack(E_l, axis=1)[:, :, None, :]

    ABW = jnp.concatenate([A, B, W[:, :, None, :]], axis=2)
    P = _sc_products(ABW.reshape(N_FLAT, 2 * len(S) + 1, NB)).reshape(
        2, N_BLK, N_PAIR, NB
    )

    return pl.pallas_call(
        _band_kernel,
        grid=(2, N_BLK // PER_STEP),
        in_specs=[
            pl.BlockSpec((1, PER_STEP, N_PAIR, NB), lambda b, k: (b, k, 0, 0)),
            pl.BlockSpec((1, PER_STEP, 1, NB), lambda b, k: (b, k, 0, 0)),
        ],
        out_specs=pl.BlockSpec((1, PER_STEP, NB, NB), lambda b, k: (b, k, 0, 0)),
        out_shape=jax.ShapeDtypeStruct((2, N_BLK, NB, NB), jnp.float32),
        compiler_params=pltpu.CompilerParams(
            dimension_semantics=("parallel", "parallel")
        ),
    )(P, E)


# speedup vs baseline: 1.0215x; 1.0139x over previous
import jax
import jax.numpy as jnp
import numpy as np
from jax import lax
from jax.experimental import pallas as pl
from jax.experimental.pallas import tpu as pltpu
from jax.experimental.pallas import tpu_sc as plsc

N_T, N_Y, N_X = 8, 32, 32
NB = N_X * N_Y
OFFS = (-64, -33, -32, -31, -2, -1, 0, 1, 2, 31, 32, 33, 64)
S = (-32, -1, 0, 1, 32)
N_BLK = 3 * N_T - 2
N_FLAT = 2 * N_BLK
N_PAIR = len(S) * len(S)

SC_CORES = 2
SC_SUBCORES = 16
SC_LANES = 16
N_WORK = SC_CORES * SC_SUBCORES

PER_STEP = 2
SUB = 128


def _np_masks():
    k = np.arange(NB)
    x = k % N_X
    y = k // N_X
    me = ((x + 1) < N_X).astype(np.float32)
    mw = ((x - 1) >= 0).astype(np.float32)
    mn = ((y + 1) < N_Y).astype(np.float32)
    ms = ((y - 1) >= 0).astype(np.float32)
    return me, mw, mn, ms


_ME, _MW, _MN, _MS = _np_masks()


def _shift_lanes(v, o):
    if o == 0:
        return v
    z = jnp.zeros((1, abs(o)), v.dtype)
    if o > 0:
        return jnp.concatenate([z, v[:, : NB - o]], axis=1)
    return jnp.concatenate([v[:, -o:], z], axis=1)


def _sc_products_body(abw_hbm, p_hbm, abw_v, p_v):
    wid = lax.axis_index("s") * SC_CORES + lax.axis_index("c")
    half = NB // 2

    def do_task(task):
        blk = task // 2
        col0 = (task % 2) * half
        pltpu.sync_copy(abw_hbm.at[blk, :, pl.ds(col0, half)], abw_v)
        for c in range(half // SC_LANES):
            sl = pl.ds(c * SC_LANES, SC_LANES)
            wv = abw_v[2 * len(S), sl]
            for i1 in range(len(S)):
                aw = abw_v[i1, sl] * wv
                for i2 in range(len(S)):
                    p_v[i1 * len(S) + i2, sl] = aw * abw_v[len(S) + i2, sl]
        pltpu.sync_copy(p_v, p_hbm.at[blk, :, pl.ds(col0, half)])

    n_task = 2 * N_FLAT
    do_task(wid)
    do_task(wid + N_WORK)

    @pl.when(wid + 2 * N_WORK < n_task)
    def _():
        do_task(wid + 2 * N_WORK)


def _sc_products(abw_flat):
    mesh = plsc.VectorSubcoreMesh(
        core_axis_name="c", subcore_axis_name="s",
        num_cores=SC_CORES, num_subcores=SC_SUBCORES,
    )
    return pl.kernel(
        _sc_products_body,
        out_type=jax.ShapeDtypeStruct((N_FLAT, N_PAIR, NB), jnp.float32),
        mesh=mesh,
        scratch_types=[
            pltpu.VMEM((2 * len(S) + 1, NB // 2), jnp.float32),
            pltpu.VMEM((N_PAIR, NB // 2), jnp.float32),
        ],
    )(abw_flat)


def _band_kernel(p_ref, e_ref, out_ref):
  for kk in range(PER_STEP):
    dd = {d: None for d in OFFS}
    for i1, o1 in enumerate(S):
        for i2, o2 in enumerate(S):
            row = i1 * len(S) + i2
            term = _shift_lanes(p_ref[0, kk, row : row + 1, :], o1)
            d = o2 - o1
            dd[d] = term if dd[d] is None else dd[d] + term
    dd[0] = dd[0] + e_ref[0, kk]
    g = {d: _shift_lanes(dd[d], d) for d in OFFS}
    nsub = NB // SUB
    jr = jax.lax.broadcasted_iota(jnp.int32, (SUB, SUB), 1) - jax.lax.broadcasted_iota(
        jnp.int32, (SUB, SUB), 0
    )
    zero = jnp.zeros((SUB, SUB), jnp.float32)
    for sr in range(nsub):
        for sc in range(nsub):
            delta = sc - sr
            if abs(delta) > 1:
                out_ref[0, kk, sr * SUB : (sr + 1) * SUB, sc * SUB : (sc + 1) * SUB] = zero
                continue
            acc = zero
            for d in OFFS:
                c = d - SUB * delta
                if c <= -SUB or c >= SUB:
                    continue
                gd = g[d][:, sc * SUB : (sc + 1) * SUB]
                acc = jnp.where(jr == c, jnp.broadcast_to(gd, (SUB, SUB)), acc)
            out_ref[0, kk, sr * SUB : (sr + 1) * SUB, sc * SUB : (sc + 1) * SUB] = acc


def kernel(kappa, m, H, tau):
    del H
    kap = kappa[0]
    t = jnp.squeeze(tau, axis=1)
    qt = jnp.transpose(1.0 / (t * t), (0, 2, 1))
    m1 = jnp.transpose(m[:, 0], (0, 2, 1))
    m2 = jnp.transpose(m[:, 1], (0, 2, 1))
    u1 = 0.5 * m1 * _ME
    l1 = -0.5 * m1 * _MW
    u32 = 0.5 * m2 * _MN
    l32 = -0.5 * m2 * _MS
    k2 = kap * kap
    dvec = jnp.concatenate(
        [jnp.full((2, 1, NB), k2), jnp.full((2, N_T - 1, NB), 1.0 + k2)], axis=1
    )
    Md = jnp.stack([l32, l1, dvec, u1, u32], axis=2)

    ones = jnp.ones((2, NB), jnp.float32)
    zcol = jnp.zeros((2, NB), jnp.float32)
    e0 = jnp.zeros((2, 5, NB), jnp.float32).at[:, 2, :].set(1.0)

    A_l, B_l, W_l, E_l = [], [], [], []

    def add(a, b, w, e):
        A_l.append(a)
        B_l.append(b)
        W_l.append(w)
        E_l.append(e)

    add(Md[:, 0], Md[:, 0], ones, 1.05 * ones)
    add(e0, Md[:, 1], -qt[:, 1], zcol)
    for i in range(1, N_T - 1):
        add(Md[:, i], e0, -qt[:, i], zcol)
        add(Md[:, i], Md[:, i], qt[:, i], qt[:, i])
        add(e0, Md[:, i + 1], -qt[:, i + 1], zcol)
    add(Md[:, N_T - 1], e0, -qt[:, N_T - 1], zcol)
    add(Md[:, N_T - 1], Md[:, N_T - 1], qt[:, N_T - 1], zcol)

    A = jnp.stack(A_l, axis=1)
    B = jnp.stack(B_l, axis=1)
    W = jnp.stack(W_l, axis=1)
    E = jnp.stack(E_l, axis=1)[:, :, None, :]

    ABW = jnp.concatenate([A, B, W[:, :, None, :]], axis=2)
    P = _sc_products(ABW.reshape(N_FLAT, 2 * len(S) + 1, NB)).reshape(
        2, N_BLK, N_PAIR, NB
    )

    return pl.pallas_call(
        _band_kernel,
        grid=(2, N_BLK // PER_STEP),
        in_specs=[
            pl.BlockSpec((1, PER_STEP, N_PAIR, NB), lambda b, k: (b, k, 0, 0)),
            pl.BlockSpec((1, PER_STEP, 1, NB), lambda b, k: (b, k, 0, 0)),
        ],
        out_specs=pl.BlockSpec((1, PER_STEP, NB, NB), lambda b, k: (b, k, 0, 0)),
        out_shape=jax.ShapeDtypeStruct((2, N_BLK, NB, NB), jnp.float32),
        compiler_params=pltpu.CompilerParams(
            dimension_semantics=("parallel", "parallel")
        ),
    )(P, E)
